# Initial kernel scaffold; baseline (speedup 1.0000x reference)
import jax
import jax.numpy as jnp
from jax.experimental import pallas as pl


def kernel(x, table):
    raise NotImplementedError("write your pallas kernel here")


# speedup vs baseline: 1.6934x; 1.6934x over previous
import jax
import jax.numpy as jnp
from jax.experimental import pallas as pl

N, S, E = 4, 4096, 1024
BS = 512


def _body(x_ref, t_ref, o_ref):
    o_ref[0] = x_ref[0] + t_ref[...]


def kernel(x, table):
    return pl.pallas_call(
        _body,
        grid=(S // BS, N),
        in_specs=[
            pl.BlockSpec((1, BS, E), lambda s, n: (n, s, 0)),
            pl.BlockSpec((BS, E), lambda s, n: (s, 0)),
        ],
        out_specs=pl.BlockSpec((1, BS, E), lambda s, n: (n, s, 0)),
        out_shape=jax.ShapeDtypeStruct((N, S, E), jnp.float32),
    )(x, table)
